# baseline (speedup 1.0000x reference)
import jax
import jax.numpy as jnp
from jax import lax
from jax.experimental import pallas as pl
from jax.experimental.pallas import tpu as pltpu
from jax.experimental.pallas import tpu_sc as plsc

B = 4096
D = 64
L = 16
VU = 100000
VC = 1000

_info = plsc.get_sparse_core_info()
_NC, _NS = _info.num_cores, _info.num_subcores
_NW = _NC * _NS
_BPW = B // _NW

_PC = 2048


def _pack_body(inT_ref, out_ref):
    x = inT_ref[...]
    h = _PC // 2
    out_ref[:, 0:D] = x[:, 0:h].T
    out_ref[:, D:2 * D] = x[:, h:_PC].T


@jax.jit
def _pack(tableT):
    v = tableT.shape[1]
    steps = (v + _PC - 1) // _PC
    return pl.pallas_call(
        _pack_body,
        grid=(steps,),
        in_specs=[pl.BlockSpec((D, _PC), lambda i: (0, i))],
        out_specs=pl.BlockSpec((_PC // 2, 2 * D), lambda i: (i, 0)),
        out_shape=jax.ShapeDtypeStruct((steps * (_PC // 2), 2 * D),
                                       jnp.float32),
        compiler_params=pltpu.CompilerParams(
            dimension_semantics=("arbitrary",),
        ),
    )(tableT)


def _extract(idv, rr):
    return idv[pl.ds(rr, L)][0]


def _sc_user_body(ids, packed, out, idv, pairb, ob, sem):
    w = lax.axis_index("s") * _NC + lax.axis_index("c")
    base = w * _BPW
    sl = pl.ds(base, _BPW)
    pltpu.sync_copy(ids.at[sl], idv.at[pl.ds(0, _BPW)])

    def fire(rr, carry):
        i = _extract(idv, rr)
        p = (lax.shift_right_logical(i, 11) * (_PC // 2)) + (i & (_PC // 2 - 1))
        pltpu.async_copy(packed.at[p], pairb.at[rr], sem)
        return carry

    def drain(rr, carry):
        pltpu.make_async_copy(packed.at[0], pairb.at[rr], sem).wait()
        return carry

    def select(rr, carry):
        off = (lax.shift_right_logical(_extract(idv, rr), 10) & 1) * D
        for c in range(D // L):
            ob[rr, pl.ds(c * L, L)] = pairb[rr, pl.ds(off + c * L, L)]
        return carry

    lax.fori_loop(0, _BPW, fire, 0)
    lax.fori_loop(0, _BPW, drain, 0)
    lax.fori_loop(0, _BPW, select, 0)
    pltpu.sync_copy(ob, out.at[sl])


def _sc_item_cat_body(iids, cids, packed, cat_tbl, out,
                      idv, idv2, pairb, obc, ob, sem, semc):
    w = lax.axis_index("s") * _NC + lax.axis_index("c")
    base = w * _BPW
    sl = pl.ds(base, _BPW)
    pltpu.sync_copy(iids.at[sl], idv.at[pl.ds(0, _BPW)])
    pltpu.sync_copy(cids.at[sl], idv2.at[pl.ds(0, _BPW)])

    def fire(rr, carry):
        i = _extract(idv, rr)
        p = (lax.shift_right_logical(i, 11) * (_PC // 2)) + (i & (_PC // 2 - 1))
        pltpu.async_copy(packed.at[p], pairb.at[rr], sem)
        pltpu.async_copy(cat_tbl.at[_extract(idv2, rr)], obc.at[rr], semc)
        return carry

    def drain(rr, carry):
        pltpu.make_async_copy(packed.at[0], pairb.at[rr], sem).wait()
        pltpu.make_async_copy(cat_tbl.at[0], obc.at[rr], semc).wait()
        return carry

    def select(rr, carry):
        off = (lax.shift_right_logical(_extract(idv, rr), 10) & 1) * D
        for c in range(D // L):
            cs = pl.ds(c * L, L)
            ob[rr, cs] = pairb[rr, pl.ds(off + c * L, L)] + obc[rr, cs]
        return carry

    lax.fori_loop(0, _BPW, fire, 0)
    lax.fori_loop(0, _BPW, drain, 0)
    lax.fori_loop(0, _BPW, select, 0)
    pltpu.sync_copy(ob, out.at[sl])


_MESH = dict(core_axis_name="c", subcore_axis_name="s")


@jax.jit
def _sc_user(ids, packed):
    f = pl.kernel(
        _sc_user_body,
        out_type=jax.ShapeDtypeStruct((B, D), jnp.float32),
        mesh=plsc.VectorSubcoreMesh(**_MESH),
        scratch_types=[
            pltpu.VMEM((_BPW + L,), jnp.int32),
            pltpu.VMEM((_BPW, 2 * D), jnp.float32),
            pltpu.VMEM((_BPW, D), jnp.float32),
            pltpu.SemaphoreType.DMA,
        ],
        compiler_params=pltpu.CompilerParams(use_tc_tiling_on_sc=True),
    )
    return f(ids, packed)


@jax.jit
def _sc_item_cat(iids, cids, packed, cat_tbl):
    f = pl.kernel(
        _sc_item_cat_body,
        out_type=jax.ShapeDtypeStruct((B, D), jnp.float32),
        mesh=plsc.VectorSubcoreMesh(**_MESH),
        scratch_types=[
            pltpu.VMEM((_BPW + L,), jnp.int32),
            pltpu.VMEM((_BPW + L,), jnp.int32),
            pltpu.VMEM((_BPW, 2 * D), jnp.float32),
            pltpu.VMEM((_BPW, D), jnp.float32),
            pltpu.VMEM((_BPW, D), jnp.float32),
            pltpu.SemaphoreType.DMA,
            pltpu.SemaphoreType.DMA,
        ],
        compiler_params=pltpu.CompilerParams(use_tc_tiling_on_sc=True),
    )
    return f(iids, cids, packed, cat_tbl)


def _tc_body(ttT_ref, tagsT_ref, ipc_ref, out_ref):
    acc = jnp.dot(ttT_ref[...], tagsT_ref[...],
                  preferred_element_type=jnp.float32)
    out_ref[...] = acc + ipc_ref[...].T


_BN = 512


@jax.jit
def _tc_matmul_add(ttT, tagsT, ipc):
    k = ttT.shape[1]
    return pl.pallas_call(
        _tc_body,
        grid=(B // _BN,),
        in_specs=[
            pl.BlockSpec((D, k), lambda i: (0, 0)),
            pl.BlockSpec((k, _BN), lambda i: (0, i)),
            pl.BlockSpec((_BN, D), lambda i: (i, 0)),
        ],
        out_specs=pl.BlockSpec((D, _BN), lambda i: (0, i)),
        out_shape=jax.ShapeDtypeStruct((D, B), jnp.float32),
        compiler_params=pltpu.CompilerParams(
            dimension_semantics=("arbitrary",),
        ),
    )(ttT, tagsT, ipc)


def kernel(user_ids, item_ids, attr_category, attr_tags,
           user_table, item_table, category_table, tags_table):
    uids = user_ids.astype(jnp.int32)
    iids = item_ids.astype(jnp.int32)
    cids = attr_category.astype(jnp.int32)
    packed_item = _pack(item_table.T)
    packed_user = _pack(user_table.T)
    ipc = _sc_item_cat(iids, cids, packed_item, category_table)
    user_emb = _sc_user(uids, packed_user)
    item_totalT = _tc_matmul_add(tags_table.T, attr_tags.T, ipc)
    return (user_emb, item_totalT.T)
